# Initial kernel scaffold; baseline (speedup 1.0000x reference)
import jax
import jax.numpy as jnp
from jax.experimental import pallas as pl


def kernel(input_features, input_features_global, batch_ids):
    raise NotImplementedError("write your pallas kernel here")


# speedup vs baseline: 1.8569x; 1.8569x over previous
import functools

import jax
import jax.numpy as jnp
from jax import lax
from jax.experimental import pallas as pl
from jax.experimental.pallas import tpu as pltpu
from jax.experimental.pallas import tpu_sc as plsc

N = 1_000_000
D = 64
B = 16
L = 16
NW = 32

C = 512
M = N // C
TAIL = N - M * C
TRIPS = -(-M // NW)


def _run(in_hbm, glob_hbm, ids_hbm, out_hbm, ids_v, buf_v, glob_v):
    c = lax.axis_index("c")
    s = lax.axis_index("s")
    wid = s * 2 + c

    pltpu.sync_copy(glob_hbm, glob_v)

    def process(base, rows):
        pltpu.sync_copy(ids_hbm.at[pl.ds(base, rows)], ids_v.at[pl.ds(0, rows)])
        pltpu.sync_copy(in_hbm.at[pl.ds(base, rows), :], buf_v.at[pl.ds(0, rows), :])
        first = ids_v[pl.ds(0, L)][0]
        last = ids_v[pl.ds(rows - L, L)][L - 1]

        @pl.when(first == last)
        def _fast():
            g = [glob_v[first, pl.ds(j * L, L)] for j in range(D // L)]

            def row(i, carry):
                for j in range(D // L):
                    plsc.addupdate(buf_v.at[i, pl.ds(j * L, L)], g[j])
                return carry

            lax.fori_loop(0, rows, row, 0)

        @pl.when(first != last)
        def _mixed():
            def grp(i0, carry):
                idvec = ids_v[pl.ds(i0, L)]
                for l in range(L):
                    b = idvec[l]
                    for j in range(D // L):
                        plsc.addupdate(
                            buf_v.at[i0 + l, pl.ds(j * L, L)],
                            glob_v[b, pl.ds(j * L, L)],
                        )
                return carry

            lax.fori_loop(0, rows // L, lambda t, c: grp(t * L, c), 0)

        pltpu.sync_copy(buf_v.at[pl.ds(0, rows), :], out_hbm.at[pl.ds(base, rows), :])

    def trip(t, carry):
        k = wid + t * NW

        @pl.when(k < M)
        def _():
            process(k * C, C)

        return carry

    lax.fori_loop(0, TRIPS, trip, 0)

    if TAIL:
        @pl.when(wid == NW - 1)
        def _tail():
            process(M * C, TAIL)


def kernel(input_features, input_features_global, batch_ids):
    ids = batch_ids.astype(jnp.int32)
    mesh = plsc.VectorSubcoreMesh(core_axis_name="c", subcore_axis_name="s")
    run = functools.partial(
        pl.kernel,
        mesh=mesh,
        out_type=jax.ShapeDtypeStruct((N, D), jnp.float32),
        scratch_types=[
            pltpu.VMEM((C,), jnp.int32),
            pltpu.VMEM((C, D), jnp.float32),
            pltpu.VMEM((B, D), jnp.float32),
        ],
    )(_run)
    return run(input_features, input_features_global, ids)
